# Initial kernel scaffold; baseline (speedup 1.0000x reference)
import jax
import jax.numpy as jnp
from jax.experimental import pallas as pl


def kernel(x, token_table, pos_table):
    raise NotImplementedError("write your pallas kernel here")


# speedup vs baseline: 4.2924x; 4.2924x over previous
import functools

import jax
import jax.numpy as jnp
from jax import lax
from jax.experimental import pallas as pl
from jax.experimental.pallas import tpu as pltpu
from jax.experimental.pallas import tpu_sc as plsc

_VOCAB = 100000
_MAXLEN = 200
_DIM = 32
_BATCH = 4096

_NC = 2
_NS = 16
_NW = _NC * _NS

_ROWS_PER_W = _BATCH // _NW
_C = 4
_TOK = _C * _MAXLEN
_NCHUNK = _ROWS_PER_W // _C
_SPAN = _ROWS_PER_W * _MAXLEN


def _embed_kernel(x_hbm, tok_hbm, pos_hbm, out_hbm, idx_v, rows_v, posc_v, sem):
    wid = lax.axis_index("s") * _NC + lax.axis_index("c")
    base = wid * _SPAN

    for r in range(_C):
        pltpu.sync_copy(pos_hbm, posc_v.at[pl.ds(r * _MAXLEN, _MAXLEN)])

    def chunk_body(g, carry):
        tok0 = base + g * _TOK
        pltpu.sync_copy(x_hbm.at[pl.ds(tok0, _TOK)], idx_v)
        pltpu.async_copy(tok_hbm.at[idx_v], rows_v, sem).wait()

        def add_row(t, c):
            p0 = posc_v[t, pl.ds(0, 16)]
            p1 = posc_v[t, pl.ds(16, 16)]
            plsc.addupdate(rows_v.at[t, pl.ds(0, 16)], p0)
            plsc.addupdate(rows_v.at[t, pl.ds(16, 16)], p1)
            return c

        lax.fori_loop(0, _TOK, add_row, 0)
        pltpu.sync_copy(rows_v, out_hbm.at[pl.ds(tok0, _TOK)])
        return carry

    lax.fori_loop(0, _NCHUNK, chunk_body, 0)


@functools.partial(jax.jit, static_argnames=())
def kernel(x, token_table, pos_table):
    b, l = x.shape
    xf = x.reshape(-1).astype(jnp.int32)
    mesh = plsc.VectorSubcoreMesh(core_axis_name="c", subcore_axis_name="s")
    run = functools.partial(
        pl.kernel,
        mesh=mesh,
        compiler_params=pltpu.CompilerParams(use_tc_tiling_on_sc=False),
        out_type=jax.ShapeDtypeStruct((b * l, _DIM), jnp.float32),
        scratch_types=[
            pltpu.VMEM((_TOK,), jnp.int32),
            pltpu.VMEM((_TOK, _DIM), jnp.float32),
            pltpu.VMEM((_TOK, _DIM), jnp.float32),
            pltpu.SemaphoreType.DMA,
        ],
    )(_embed_kernel)
    out = run(xf, token_table, pos_table)
    return out.reshape(b, l, _DIM)
